# baseline (speedup 1.0000x reference)
import jax
import jax.numpy as jnp
from jax.experimental import pallas as pl
from jax.experimental.pallas import tpu as pltpu


def _body(tt_ref, x_ref, w_ref, o_ref):
    t = tt_ref[0]
    w = w_ref[...]
    lane = jax.lax.broadcasted_iota(jnp.int32, w.shape, 1)
    col = jnp.sum(jnp.where(lane == t, w, 0.0), axis=1, keepdims=True)
    o_ref[...] = x_ref[...] - col


def kernel(x, weight, token_type):
    n, d = x.shape
    e = weight.shape[1]
    bn = 1024
    tt = jnp.asarray(token_type, jnp.int32).reshape(1)
    return pl.pallas_call(
        _body,
        grid=(n // bn,),
        in_specs=[
            pl.BlockSpec(memory_space=pltpu.SMEM),
            pl.BlockSpec((bn, d), lambda i: (i, 0)),
            pl.BlockSpec((bn, e), lambda i: (i, 0)),
        ],
        out_specs=pl.BlockSpec((bn, d), lambda i: (i, 0)),
        out_shape=jax.ShapeDtypeStruct((n, d), jnp.float32),
    )(tt, x, weight)


# speedup vs baseline: 15.3017x; 15.3017x over previous
import jax
import jax.numpy as jnp
from jax.experimental import pallas as pl
from jax.experimental.pallas import tpu as pltpu


def _body(tt_ref, x_ref, w_ref, o_ref):
    t = tt_ref[0]
    w = w_ref[...]
    lane = jax.lax.broadcasted_iota(jnp.int32, w.shape, 1)
    col = jnp.sum(jnp.where(lane == t, w, 0.0), axis=1, keepdims=True)
    o_ref[...] = x_ref[...] - col


def kernel(x, weight, token_type):
    n, d = x.shape
    e = weight.shape[1]
    bn = 1024
    tt = jnp.asarray(token_type, jnp.int32).reshape(1)
    return pl.pallas_call(
        _body,
        grid=(n // bn,),
        in_specs=[
            pl.BlockSpec(memory_space=pltpu.SMEM),
            pl.BlockSpec((bn, d), lambda i: (i, 0)),
            pl.BlockSpec((bn, e), lambda i: (i, 0)),
        ],
        out_specs=pl.BlockSpec((bn, d), lambda i: (i, 0)),
        out_shape=jax.ShapeDtypeStruct((n, d), jnp.float32),
    )(tt, x, weight[:n])
